# baseline (speedup 1.0000x reference)
import functools

import jax
import jax.numpy as jnp
from jax.experimental import pallas as pl
from jax.experimental.pallas import tpu as pltpu

HIDDEN = 1024
EPS = 1e-12
ROWS_PER_STEP = 8


def _ln_kernel(ids_ref, pos_ref, gamma_ref, beta_ref, *refs):
    word_refs = refs[:ROWS_PER_STEP]
    out_ref = refs[ROWS_PER_STEP]
    emb = jnp.concatenate([r[0] for r in word_refs], axis=0) + pos_ref[...]
    mean = jnp.mean(emb, axis=-1, keepdims=True)
    ctr = emb - mean
    var = jnp.mean(ctr * ctr, axis=-1, keepdims=True)
    normed = ctr * jax.lax.rsqrt(var + EPS)
    out_ref[...] = normed * gamma_ref[...] + beta_ref[...]


def kernel(input_ids, word_table, pos_table, gamma, beta):
    B, S = input_ids.shape
    n_tokens = B * S
    n_steps = n_tokens // ROWS_PER_STEP
    blocks_per_seq = S // ROWS_PER_STEP

    ids_flat = input_ids.reshape(n_tokens).astype(jnp.int32)
    gamma2 = gamma.reshape(1, HIDDEN)
    beta2 = beta.reshape(1, HIDDEN)
    word_table3 = word_table.reshape(word_table.shape[0], 1, HIDDEN)

    def word_map(j):
        def index_map(i, ids_ref):
            return (ids_ref[i * ROWS_PER_STEP + j], 0, 0)
        return index_map

    grid_spec = pltpu.PrefetchScalarGridSpec(
        num_scalar_prefetch=1,
        grid=(n_steps,),
        in_specs=[
            pl.BlockSpec((ROWS_PER_STEP, HIDDEN),
                         lambda i, ids_ref: (i % blocks_per_seq, 0)),
            pl.BlockSpec((1, HIDDEN), lambda i, ids_ref: (0, 0)),
            pl.BlockSpec((1, HIDDEN), lambda i, ids_ref: (0, 0)),
        ] + [
            pl.BlockSpec((1, 1, HIDDEN), word_map(j)) for j in range(ROWS_PER_STEP)
        ],
        out_specs=pl.BlockSpec((ROWS_PER_STEP, HIDDEN), lambda i, ids_ref: (i, 0)),
    )

    out_flat = pl.pallas_call(
        _ln_kernel,
        grid_spec=grid_spec,
        out_shape=jax.ShapeDtypeStruct((n_tokens, HIDDEN), jnp.float32),
    )(ids_flat, pos_table, gamma2, beta2, *([word_table3] * ROWS_PER_STEP))

    return out_flat.reshape(B, S, HIDDEN)


# speedup vs baseline: 8.9784x; 8.9784x over previous
import functools

import jax
import jax.numpy as jnp
from jax import lax
from jax.experimental import pallas as pl
from jax.experimental.pallas import tpu as pltpu
from jax.experimental.pallas import tpu_sc as plsc

HIDDEN = 1024
SEQ = 512
EPS = 1e-12

NUM_WORKERS = 32
CHUNK = 16
NBUF = 4
LN_ROWS = 256


def _sc_gather(ids_flat, word_table):
    n = ids_flat.shape[0]
    b_per_w = n // NUM_WORKERS
    nchunks = b_per_w // CHUNK
    mesh = plsc.VectorSubcoreMesh(core_axis_name="c", subcore_axis_name="s")

    @functools.partial(
        pl.kernel,
        out_type=jax.ShapeDtypeStruct((n, HIDDEN), jnp.float32),
        mesh=mesh,
        scratch_types=(
            [pltpu.VMEM((b_per_w,), jnp.int32)]
            + [pltpu.VMEM((CHUNK, HIDDEN), jnp.float32) for _ in range(NBUF)]
            + [pltpu.SemaphoreType.DMA for _ in range(2 * NBUF)]
        ),
    )
    def gather_kernel(table_hbm, ids_hbm, out_hbm, idx_v, *scratch):
        bufs = scratch[:NBUF]
        gsems = scratch[NBUF:2 * NBUF]
        wsems = scratch[2 * NBUF:]
        wid = lax.axis_index("s") * 2 + lax.axis_index("c")
        base = wid * b_per_w

        pltpu.sync_copy(ids_hbm.at[pl.ds(base, b_per_w)], idx_v)

        def start_gather(b, c):
            pltpu.async_copy(
                table_hbm.at[idx_v.at[pl.ds(c * CHUNK, CHUNK)]],
                bufs[b], gsems[b])

        def wait_gather(b, c):
            pltpu.make_async_copy(
                table_hbm.at[idx_v.at[pl.ds(c * CHUNK, CHUNK)]],
                bufs[b], gsems[b]).wait()

        def start_write(b, c):
            pltpu.async_copy(
                bufs[b], out_hbm.at[pl.ds(base + c * CHUNK, CHUNK)],
                wsems[b])

        def wait_write(b, c):
            pltpu.make_async_copy(
                bufs[b], out_hbm.at[pl.ds(base + c * CHUNK, CHUNK)],
                wsems[b]).wait()

        for b in range(NBUF):
            start_gather(b, b)

        @pl.loop(0, nchunks, step=NBUF)
        def _(c0):
            for b in range(NBUF):
                c = c0 + b
                wait_gather(b, c)
                start_write(b, c)

                @pl.when(c0 + NBUF < nchunks)
                def _():
                    wait_write(b, c)
                    start_gather(b, c + NBUF)

        for b in range(NBUF):
            wait_write(b, nchunks - NBUF + b)

    return gather_kernel(word_table, ids_flat)


def _ln_kernel(emb_ref, pos_ref, gamma_ref, beta_ref, out_ref):
    emb = emb_ref[...] + pos_ref[...]
    mean = jnp.mean(emb, axis=-1, keepdims=True)
    ctr = emb - mean
    var = jnp.mean(ctr * ctr, axis=-1, keepdims=True)
    out_ref[...] = (ctr * lax.rsqrt(var + EPS)) * gamma_ref[...] + beta_ref[...]


def _tc_layernorm(gathered, pos_table, gamma, beta):
    n = gathered.shape[0]
    blocks_per_seq = SEQ // LN_ROWS
    grid = (n // LN_ROWS,)
    return pl.pallas_call(
        _ln_kernel,
        grid=grid,
        in_specs=[
            pl.BlockSpec((LN_ROWS, HIDDEN), lambda i: (i, 0)),
            pl.BlockSpec((LN_ROWS, HIDDEN), lambda i: (i % blocks_per_seq, 0)),
            pl.BlockSpec((1, HIDDEN), lambda i: (0, 0)),
            pl.BlockSpec((1, HIDDEN), lambda i: (0, 0)),
        ],
        out_specs=pl.BlockSpec((LN_ROWS, HIDDEN), lambda i: (i, 0)),
        out_shape=jax.ShapeDtypeStruct((n, HIDDEN), jnp.float32),
    )(gathered, pos_table, gamma.reshape(1, HIDDEN), beta.reshape(1, HIDDEN))


def kernel(input_ids, word_table, pos_table, gamma, beta):
    B, S = input_ids.shape
    ids_flat = input_ids.reshape(B * S).astype(jnp.int32)
    gathered = _sc_gather(ids_flat, word_table)
    out = _tc_layernorm(gathered, pos_table, gamma, beta)
    return out.reshape(B, S, HIDDEN)


# speedup vs baseline: 11.2796x; 1.2563x over previous
import functools

import jax
import jax.numpy as jnp
from jax import lax
from jax.experimental import pallas as pl
from jax.experimental.pallas import tpu as pltpu
from jax.experimental.pallas import tpu_sc as plsc

HIDDEN = 1024
SEQ = 512
EPS = 1e-12

NUM_WORKERS = 32
CHUNK = 16
NBUF = 4
LN_ROWS = 256


def _sc_gather(ids_flat, word_table):
    n = ids_flat.shape[0]
    b_per_w = n // NUM_WORKERS
    nchunks = b_per_w // CHUNK
    mesh = plsc.VectorSubcoreMesh(core_axis_name="c", subcore_axis_name="s")

    @functools.partial(
        pl.kernel,
        out_type=jax.ShapeDtypeStruct((n, HIDDEN), jnp.float32),
        mesh=mesh,
        scratch_types=(
            [pltpu.VMEM((b_per_w,), jnp.int32)]
            + [pltpu.VMEM((CHUNK, HIDDEN), jnp.float32) for _ in range(NBUF)]
            + [pltpu.SemaphoreType.DMA for _ in range(2 * NBUF)]
        ),
    )
    def gather_kernel(table_hbm, ids_hbm, out_hbm, idx_v, *scratch):
        bufs = scratch[:NBUF]
        gsems = scratch[NBUF:2 * NBUF]
        wsems = scratch[2 * NBUF:]
        wid = lax.axis_index("s") * 2 + lax.axis_index("c")
        base = wid * b_per_w

        pltpu.sync_copy(ids_hbm.at[pl.ds(base, b_per_w)], idx_v)

        def start_gather(b, c):
            pltpu.async_copy(
                table_hbm.at[idx_v.at[pl.ds(c * CHUNK, CHUNK)]],
                bufs[b], gsems[b])

        def wait_gather(b, c):
            pltpu.make_async_copy(
                table_hbm.at[idx_v.at[pl.ds(c * CHUNK, CHUNK)]],
                bufs[b], gsems[b]).wait()

        def start_write(b, c):
            pltpu.async_copy(
                bufs[b], out_hbm.at[pl.ds(base + c * CHUNK, CHUNK)],
                wsems[b])

        def wait_write(b, c):
            pltpu.make_async_copy(
                bufs[b], out_hbm.at[pl.ds(base + c * CHUNK, CHUNK)],
                wsems[b]).wait()

        for b in range(NBUF):
            start_gather(b, b)

        @pl.loop(0, nchunks, step=NBUF)
        def _(c0):
            for b in range(NBUF):
                c = c0 + b
                wait_gather(b, c)
                start_write(b, c)

                @pl.when(c0 + NBUF < nchunks)
                def _():
                    wait_write(b, c)
                    start_gather(b, c + NBUF)

        for b in range(NBUF):
            wait_write(b, nchunks - NBUF + b)

    return gather_kernel(word_table, ids_flat)


def _ln_kernel(emb_ref, pos_ref, gamma_ref, beta_ref, out_ref):
    emb = emb_ref[...] + pos_ref[...]
    mean = jnp.mean(emb, axis=-1, keepdims=True)
    ctr = emb - mean
    var = jnp.mean(ctr * ctr, axis=-1, keepdims=True)
    out_ref[...] = (ctr * lax.rsqrt(var + EPS)) * gamma_ref[...] + beta_ref[...]


def _tc_layernorm(gathered, pos_table, gamma, beta):
    n = gathered.shape[0]
    grid = (n // SEQ,)
    return pl.pallas_call(
        _ln_kernel,
        grid=grid,
        in_specs=[
            pl.BlockSpec((SEQ, HIDDEN), lambda i: (i, 0)),
            pl.BlockSpec((SEQ, HIDDEN), lambda i: (0, 0)),
            pl.BlockSpec((1, HIDDEN), lambda i: (0, 0)),
            pl.BlockSpec((1, HIDDEN), lambda i: (0, 0)),
        ],
        out_specs=pl.BlockSpec((SEQ, HIDDEN), lambda i: (i, 0)),
        out_shape=jax.ShapeDtypeStruct((n, HIDDEN), jnp.float32),
    )(gathered, pos_table, gamma.reshape(1, HIDDEN), beta.reshape(1, HIDDEN))


def kernel(input_ids, word_table, pos_table, gamma, beta):
    B, S = input_ids.shape
    ids_flat = input_ids.reshape(B * S).astype(jnp.int32)
    gathered = _sc_gather(ids_flat, word_table)
    out = _tc_layernorm(gathered, pos_table, gamma, beta)
    return out.reshape(B, S, HIDDEN)


# speedup vs baseline: 11.7247x; 1.0395x over previous
import functools

import jax
import jax.numpy as jnp
from jax import lax
from jax.experimental import pallas as pl
from jax.experimental.pallas import tpu as pltpu
from jax.experimental.pallas import tpu_sc as plsc

HIDDEN = 1024
SEQ = 512
EPS = 1e-12

NUM_WORKERS = 32
CHUNK = 16
NBUF = 4
N_PIPE = 4


def _sc_gather(ids_flat, word_table):
    n = ids_flat.shape[0]
    b_per_w = n // NUM_WORKERS
    nchunks = b_per_w // CHUNK
    mesh = plsc.VectorSubcoreMesh(core_axis_name="c", subcore_axis_name="s")

    @functools.partial(
        pl.kernel,
        out_type=jax.ShapeDtypeStruct((n, HIDDEN), jnp.float32),
        mesh=mesh,
        scratch_types=(
            [pltpu.VMEM((b_per_w,), jnp.int32)]
            + [pltpu.VMEM((CHUNK, HIDDEN), jnp.float32) for _ in range(NBUF)]
            + [pltpu.SemaphoreType.DMA for _ in range(2 * NBUF)]
        ),
    )
    def gather_kernel(table_hbm, ids_hbm, out_hbm, idx_v, *scratch):
        bufs = scratch[:NBUF]
        gsems = scratch[NBUF:2 * NBUF]
        wsems = scratch[2 * NBUF:]
        wid = lax.axis_index("s") * 2 + lax.axis_index("c")
        base = wid * b_per_w

        pltpu.sync_copy(ids_hbm.at[pl.ds(base, b_per_w)], idx_v)

        def start_gather(b, c):
            pltpu.async_copy(
                table_hbm.at[idx_v.at[pl.ds(c * CHUNK, CHUNK)]],
                bufs[b], gsems[b])

        def wait_gather(b, c):
            pltpu.make_async_copy(
                table_hbm.at[idx_v.at[pl.ds(c * CHUNK, CHUNK)]],
                bufs[b], gsems[b]).wait()

        def start_write(b, c):
            pltpu.async_copy(
                bufs[b], out_hbm.at[pl.ds(base + c * CHUNK, CHUNK)],
                wsems[b])

        def wait_write(b, c):
            pltpu.make_async_copy(
                bufs[b], out_hbm.at[pl.ds(base + c * CHUNK, CHUNK)],
                wsems[b]).wait()

        for b in range(NBUF):
            start_gather(b, b)

        @pl.loop(0, nchunks, step=NBUF)
        def _(c0):
            for b in range(NBUF):
                c = c0 + b
                wait_gather(b, c)
                start_write(b, c)

                @pl.when(c0 + NBUF < nchunks)
                def _():
                    wait_write(b, c)
                    start_gather(b, c + NBUF)

        for b in range(NBUF):
            wait_write(b, nchunks - NBUF + b)

    return gather_kernel(word_table, ids_flat)


def _ln_body(emb_ref, pos_ref, gamma_ref, beta_ref, out_ref):
    emb = emb_ref[...] + pos_ref[...]
    mean = jnp.mean(emb, axis=-1, keepdims=True)
    ctr = emb - mean
    var = jnp.mean(ctr * ctr, axis=-1, keepdims=True)
    out_ref[...] = (ctr * lax.rsqrt(var + EPS)) * gamma_ref[...] + beta_ref[...]


def _tc_layernorm_chunk(out_buf, gathered, pos_table, gamma2, beta2, row_off):
    rows = gathered.shape[0]
    seq_off = row_off // SEQ
    body = functools.partial(_ln_body)
    data_specs = [
        pl.BlockSpec((SEQ, HIDDEN), lambda i: (i, 0)),
        pl.BlockSpec((SEQ, HIDDEN), lambda i: (0, 0)),
        pl.BlockSpec((1, HIDDEN), lambda i: (0, 0)),
        pl.BlockSpec((1, HIDDEN), lambda i: (0, 0)),
    ]
    out_spec = pl.BlockSpec((SEQ, HIDDEN), lambda i: (seq_off + i, 0))
    if out_buf is None:
        n_total = N_PIPE * rows
        return pl.pallas_call(
            _ln_body,
            grid=(rows // SEQ,),
            in_specs=data_specs,
            out_specs=out_spec,
            out_shape=jax.ShapeDtypeStruct((n_total, HIDDEN), jnp.float32),
        )(gathered, pos_table, gamma2, beta2)
    n_total = out_buf.shape[0]
    return pl.pallas_call(
        lambda alias_ref, *a: _ln_body(*a),
        grid=(rows // SEQ,),
        in_specs=[pl.BlockSpec(memory_space=pl.ANY)] + data_specs,
        out_specs=out_spec,
        out_shape=jax.ShapeDtypeStruct((n_total, HIDDEN), jnp.float32),
        input_output_aliases={0: 0},
    )(out_buf, gathered, pos_table, gamma2, beta2)


def kernel(input_ids, word_table, pos_table, gamma, beta):
    B, S = input_ids.shape
    n = B * S
    ids_flat = input_ids.reshape(n).astype(jnp.int32)
    gamma2 = gamma.reshape(1, HIDDEN)
    beta2 = beta.reshape(1, HIDDEN)

    chunk_rows = n // N_PIPE
    gathered = [
        _sc_gather(lax.dynamic_slice(ids_flat, (k * chunk_rows,), (chunk_rows,)),
                   word_table)
        for k in range(N_PIPE)
    ]

    out = None
    for k in range(N_PIPE):
        out = _tc_layernorm_chunk(out, gathered[k], pos_table, gamma2, beta2,
                                  k * chunk_rows)
    return out.reshape(B, S, HIDDEN)
